# baseline (speedup 1.0000x reference)
import jax
import jax.numpy as jnp
from jax.experimental import pallas as pl
from jax.experimental.pallas import tpu as pltpu
from jax.experimental.pallas import tpu_sc as plsc

W = 256
N_WORKERS = 32
LANES = 16
NCHUNK = W // LANES


def kernel(x, emb_weight):
    batch, hist = x.shape
    vocab, emb_dim = emb_weight.shape
    n_w = batch // W
    n_tasks = hist * n_w
    per_worker = n_tasks // N_WORKERS

    tw = emb_weight.reshape(vocab // 2, 2 * emb_dim)
    idx_t = x.T.astype(jnp.int32)

    mesh = plsc.VectorSubcoreMesh(
        core_axis_name="core", subcore_axis_name="subcore"
    )

    @pl.kernel(
        out_type=jax.ShapeDtypeStruct((hist, emb_dim, batch), jnp.float32),
        mesh=mesh,
        scratch_types=[
            pltpu.VMEM((2, W), jnp.int32),
            pltpu.VMEM((2, W // 128, 128), jnp.int32),
            pltpu.VMEM((2, W), jnp.int32),
            pltpu.VMEM((2, W // 128, 128, 2 * emb_dim), jnp.float32),
            pltpu.VMEM((2, emb_dim, W), jnp.float32),
            pltpu.SemaphoreType.DMA((2,)),
            pltpu.SemaphoreType.DMA((2,)),
            pltpu.SemaphoreType.DMA((2,)),
        ],
        compiler_params=pltpu.CompilerParams(
            use_tc_tiling_on_sc=True,
            needs_layout_passes=False,
            disable_bounds_checks=True
        ),
    )
    def gather_kernel(
        tw_hbm, i_hbm, o_hbm,
        idx_v, half_v, off_v, buf_v, out_v,
        isem, gsem, osem,
    ):
        nc = jax.lax.axis_size("core")
        wid = jax.lax.axis_index("subcore") * nc + jax.lax.axis_index("core")
        t0 = wid * per_worker

        def hw(t):
            h = t // n_w
            return h, t - h * n_w

        def idx_copy(t, b):
            h, w = hw(t)
            return pltpu.make_async_copy(
                i_hbm.at[h, pl.ds(w * W, W)], idx_v.at[b], isem.at[b]
            )

        def gather_copies(b):
            return [
                pltpu.make_async_copy(
                    tw_hbm.at[half_v.at[b, p]], buf_v.at[b, p], gsem.at[b]
                )
                for p in range(W // 128)
            ]

        def out_copy(t, b):
            h, w = hw(t)
            return pltpu.make_async_copy(
                out_v.at[b], o_hbm.at[h, :, pl.ds(w * W, W)], osem.at[b]
            )

        def prep(b):
            for c in range(NCHUNK):
                s = pl.ds(c * LANES, LANES)
                v = idx_v[b, s]
                half_v[b, c // 8, pl.ds((c % 8) * LANES, LANES)] = (
                    jax.lax.shift_right_logical(v, 1)
                )
                off_v[b, s] = (v & 1) * emb_dim

        jrows = [
            jax.lax.iota(jnp.int32, LANES) + ((c % 8) * LANES)
            for c in range(NCHUNK)
        ]

        def transpose(b):
            offs = [off_v[b, pl.ds(c * LANES, LANES)] for c in range(NCHUNK)]

            @pl.loop(0, emb_dim, step=4)
            def _(d):
                for dd in range(4):
                    for c in range(NCHUNK):
                        out_v[b, d + dd, pl.ds(c * LANES, LANES)] = (
                            plsc.load_gather(
                                buf_v.at[b, c // 8],
                                [jrows[c], offs[c] + (d + dd)],
                            )
                        )

        idx_copy(t0, 0).start()
        idx_copy(t0, 0).wait()
        prep(0)
        for cp in gather_copies(0):
            cp.start()
        idx_copy(t0 + 1, 1).start()

        @pl.loop(0, per_worker, step=2)
        def _(i):
            for b in (0, 1):
                nb = 1 - b
                t = t0 + i + b

                @pl.when(i + b + 1 < per_worker)
                def _():
                    idx_copy(t + 1, nb).wait()
                    prep(nb)
                    for cp in gather_copies(nb):
                        cp.start()

                @pl.when(i + b + 2 < per_worker)
                def _():
                    idx_copy(t + 2, b).start()

                @pl.when(i + b >= 2)
                def _():
                    out_copy(t - 2, b).wait()

                for cp in gather_copies(b):
                    cp.wait()
                transpose(b)
                out_copy(t, b).start()

        out_copy(t0 + per_worker - 2, 0).wait()
        out_copy(t0 + per_worker - 1, 1).wait()

    out = gather_kernel(tw, idx_t)
    return out.transpose(2, 0, 1)


# speedup vs baseline: 1.1950x; 1.1950x over previous
import jax
import jax.numpy as jnp
from jax.experimental import pallas as pl
from jax.experimental.pallas import tpu as pltpu
from jax.experimental.pallas import tpu_sc as plsc

W = 256
N_WORKERS = 32
LANES = 16
NCHUNK = W // LANES


def kernel(x, emb_weight):
    batch, hist = x.shape
    vocab, emb_dim = emb_weight.shape
    n_w = batch // W
    n_tasks = hist * n_w
    per_worker = n_tasks // N_WORKERS

    tw = emb_weight.reshape(vocab // 2, 2 * emb_dim)
    idx_t = x.T.astype(jnp.int32)

    mesh = plsc.VectorSubcoreMesh(
        core_axis_name="core", subcore_axis_name="subcore"
    )

    @pl.kernel(
        out_type=jax.ShapeDtypeStruct((hist, emb_dim, batch), jnp.float32),
        mesh=mesh,
        scratch_types=[
            pltpu.VMEM((2, W), jnp.int32),
            pltpu.VMEM((2, W // 128, 128), jnp.int32),
            pltpu.VMEM((2, W), jnp.int32),
            pltpu.VMEM((2, W // 128, 128, 2 * emb_dim), jnp.float32),
            pltpu.VMEM((2, emb_dim, W), jnp.float32),
            pltpu.SemaphoreType.DMA((2,)),
            pltpu.SemaphoreType.DMA((2,)),
            pltpu.SemaphoreType.DMA((2,)),
        ],
        compiler_params=pltpu.CompilerParams(
            use_tc_tiling_on_sc=True,
            needs_layout_passes=False,
            disable_bounds_checks=True
        ),
    )
    def gather_kernel(
        tw_hbm, i_hbm, o_hbm,
        idx_v, half_v, off_v, buf_v, out_v,
        isem, gsem, osem,
    ):
        nc = jax.lax.axis_size("core")
        wid = jax.lax.axis_index("subcore") * nc + jax.lax.axis_index("core")
        t0 = wid * per_worker

        def hw(t):
            h = t // n_w
            return h, t - h * n_w

        def idx_copy(t, b):
            h, w = hw(t)
            return pltpu.make_async_copy(
                i_hbm.at[h, pl.ds(w * W, W)], idx_v.at[b], isem.at[b]
            )

        def gather_copies(b):
            return [
                pltpu.make_async_copy(
                    tw_hbm.at[half_v.at[b, p]], buf_v.at[b, p], gsem.at[b]
                )
                for p in range(W // 128)
            ]

        def out_copy(t, b):
            h, w = hw(t)
            return pltpu.make_async_copy(
                out_v.at[b], o_hbm.at[h, :, pl.ds(w * W, W)], osem.at[b]
            )

        def prep(b):
            for c in range(NCHUNK):
                s = pl.ds(c * LANES, LANES)
                v = idx_v[b, s]
                half_v[b, c // 8, pl.ds((c % 8) * LANES, LANES)] = (
                    jax.lax.shift_right_logical(v, 1)
                )
                off_v[b, s] = (v & 1) * emb_dim

        jrows = [
            jax.lax.iota(jnp.int32, LANES) + ((c % 8) * LANES)
            for c in range(NCHUNK)
        ]

        def transpose(b):
            offs = [off_v[b, pl.ds(c * LANES, LANES)] for c in range(NCHUNK)]

            @pl.loop(0, emb_dim, step=2)
            def _(d):
                vals = [
                    plsc.load_gather(
                        buf_v.at[b, c // 8],
                        [jrows[c], offs[c] + (d + dd)],
                    )
                    for dd in range(2)
                    for c in range(NCHUNK)
                ]
                k = 0
                for dd in range(2):
                    for c in range(NCHUNK):
                        out_v[b, d + dd, pl.ds(c * LANES, LANES)] = vals[k]
                        k += 1

        idx_copy(t0, 0).start()
        idx_copy(t0, 0).wait()
        prep(0)
        for cp in gather_copies(0):
            cp.start()
        idx_copy(t0 + 1, 1).start()

        @pl.loop(0, per_worker, step=2)
        def _(i):
            for b in (0, 1):
                nb = 1 - b
                t = t0 + i + b

                @pl.when(i + b + 1 < per_worker)
                def _():
                    idx_copy(t + 1, nb).wait()
                    prep(nb)
                    for cp in gather_copies(nb):
                        cp.start()

                @pl.when(i + b + 2 < per_worker)
                def _():
                    idx_copy(t + 2, b).start()

                @pl.when(i + b >= 2)
                def _():
                    out_copy(t - 2, b).wait()

                for cp in gather_copies(b):
                    cp.wait()
                transpose(b)
                out_copy(t, b).start()

        out_copy(t0 + per_worker - 2, 0).wait()
        out_copy(t0 + per_worker - 1, 1).wait()

    out = gather_kernel(tw, idx_t)
    return out.transpose(2, 0, 1)
